# baseline (speedup 1.0000x reference)
import functools

import jax
import jax.numpy as jnp
from jax import lax
from jax.experimental import pallas as pl
from jax.experimental.pallas import tpu as pltpu
from jax.experimental.pallas import tpu_sc as plsc

L = 16


def _build_sc_gather(N, V, D, F, NW):
    per_w = N // NW
    R = 128
    n_chunks = per_w // R
    assert per_w * NW == N and n_chunks * R == per_w and per_w % F == 0

    mesh = plsc.VectorSubcoreMesh(core_axis_name="c", subcore_axis_name="s")

    @functools.partial(
        pl.kernel,
        mesh=mesh,
        out_type=jax.ShapeDtypeStruct((N, D), jnp.float32),
        compiler_params=pltpu.CompilerParams(use_tc_tiling_on_sc=False),
        scratch_types=[
            pltpu.VMEM((per_w,), jnp.int32),
            pltpu.VMEM((per_w,), jnp.int32),
            pltpu.VMEM((R, D), jnp.float32),
            pltpu.VMEM((R, D), jnp.float32),
            pltpu.SemaphoreType.DMA,
            pltpu.SemaphoreType.DMA,
        ],
    )
    def sc_kernel(idx_hbm, table_hbm, pat_hbm, out_hbm,
                  idx_v, pat_v, buf0, buf1, sem0, sem1):
        wid = lax.axis_index("s") * 2 + lax.axis_index("c")
        base = wid * per_w

        pltpu.sync_copy(idx_hbm.at[pl.ds(base, per_w)], idx_v)
        pltpu.sync_copy(pat_hbm, pat_v)

        def add_body(i, _):
            sl = pl.ds(i * L, L)
            idx_v[sl] = idx_v[sl] + pat_v[sl]
            return 0

        lax.fori_loop(0, per_w // L, add_body, 0)

        def gather(c, buf, sem):
            pltpu.async_copy(table_hbm.at[idx_v.at[pl.ds(c * R, R)]], buf, sem)

        def gwait(buf, sem):
            pltpu.make_async_copy(table_hbm.at[pl.ds(0, R)], buf, sem).wait()

        gather(0, buf0, sem0)

        def chunk_body(j, _):
            c0 = 2 * j
            gather(c0 + 1, buf1, sem1)
            gwait(buf0, sem0)
            pltpu.sync_copy(buf0, out_hbm.at[pl.ds(base + c0 * R, R)])

            @pl.when(j + 1 < n_chunks // 2)
            def _():
                gather(c0 + 2, buf0, sem0)

            gwait(buf1, sem1)
            pltpu.sync_copy(buf1, out_hbm.at[pl.ds(base + (c0 + 1) * R, R)])
            return 0

        lax.fori_loop(0, n_chunks // 2, chunk_body, 0)

    return sc_kernel


def kernel(categorical_inputs, table, offsets):
    B, F = categorical_inputs.shape
    V, D = table.shape
    N = B * F
    NW = 32

    idx = categorical_inputs.astype(jnp.int32).reshape(N)
    per_w = N // NW
    pat = jnp.tile(offsets[:F].astype(jnp.int32), per_w // F)

    sc = _build_sc_gather(N, V, D, F, NW)
    out = sc(idx, table, pat)
    return out.reshape(B, F, D)


# speedup vs baseline: 1.1689x; 1.1689x over previous
import functools

import jax
import jax.numpy as jnp
from jax import lax
from jax.experimental import pallas as pl
from jax.experimental.pallas import tpu as pltpu
from jax.experimental.pallas import tpu_sc as plsc

L = 16
SB = 256


def _build_sc_gather(B, F, D, V, NW):
    BW = B // NW
    S = BW // SB
    G = SB // L
    DG = D // L
    NK = F * S
    assert BW * NW == B and S * SB == BW and G * L == SB and DG * L == D
    assert NK % 2 == 0

    mesh = plsc.VectorSubcoreMesh(core_axis_name="c", subcore_axis_name="s")

    @functools.partial(
        pl.kernel,
        mesh=mesh,
        compiler_params=pltpu.CompilerParams(
            needs_layout_passes=False, use_tc_tiling_on_sc=True),
        out_type=jax.ShapeDtypeStruct((F, D, B), jnp.float32),
        scratch_types=[
            pltpu.VMEM((SB,), jnp.int32),
            pltpu.VMEM((SB,), jnp.int32),
            pltpu.VMEM((F * L,), jnp.int32),
            pltpu.VMEM((SB, D), jnp.float32),
            pltpu.VMEM((SB, D), jnp.float32),
            pltpu.VMEM((L, SB), jnp.float32),
            pltpu.SemaphoreType.DMA,
            pltpu.SemaphoreType.DMA,
        ],
    )
    def sc_kernel(table_hbm, idx_hbm, off_hbm, out_hbm,
                  idxq0, idxq1, off_v, buf0, buf1, strip, sem0, sem1):
        wid = lax.axis_index("s") * 2 + lax.axis_index("c")
        b0 = wid * BW

        pltpu.sync_copy(off_hbm, off_v)

        qi = lax.iota(jnp.int32, L)

        def issue_block(k, idxq, buf, sem):
            f = k // S
            base = b0 + (k % S) * SB
            pltpu.sync_copy(idx_hbm.at[f, pl.ds(base, SB)], idxq)
            ov = off_v[pl.ds(f * L, L)]

            def gbody(g, _):
                vv = idxq[pl.ds(g * L, L)] + ov
                for j in range(L):
                    v = vv[j]
                    pltpu.async_copy(
                        table_hbm.at[pl.ds(v, 1)],
                        buf.at[pl.ds(g * L + j, 1)],
                        sem)
                return 0
            lax.fori_loop(0, G, gbody, 0)

        def drain(buf, sem):
            pltpu.make_async_copy(table_hbm.at[pl.ds(0, SB)], buf, sem).wait()

        def emit_block(k, buf):
            f = k // S
            base = b0 + (k % S) * SB

            def sbody(dg, _):
                def gbody(g, _):
                    rows = qi + g * L
                    for dl in range(L):
                        col = jnp.full((L,), dg * L + dl, jnp.int32)
                        gvec = plsc.load_gather(buf, [rows, col])
                        strip[dl, pl.ds(g * L, L)] = gvec
                    return 0
                lax.fori_loop(0, G, gbody, 0)
                pltpu.sync_copy(strip,
                                out_hbm.at[f, pl.ds(dg * L, L), pl.ds(base, SB)])
                return 0
            lax.fori_loop(0, DG, sbody, 0)

        bufs = (buf0, buf1)
        idxqs = (idxq0, idxq1)
        sems = (sem0, sem1)

        issue_block(0, idxqs[0], bufs[0], sems[0])

        def fbody(j, _):
            for p in range(2):
                k = j * 2 + p

                @pl.when(k + 1 < NK)
                def _():
                    issue_block(k + 1, idxqs[1 - p], bufs[1 - p], sems[1 - p])
                drain(bufs[p], sems[p])
                emit_block(k, bufs[p])
            return 0

        lax.fori_loop(0, NK // 2, fbody, 0)

    return sc_kernel


def kernel(categorical_inputs, table, offsets):
    B, F = categorical_inputs.shape
    V, D = table.shape
    NW = 32

    idxT = jnp.transpose(categorical_inputs).astype(jnp.int32)
    off = jnp.repeat(offsets[:F].astype(jnp.int32), L)

    sc = _build_sc_gather(B, F, D, V, NW)
    outP = sc(table, idxT, off)
    return jnp.transpose(outP, (2, 0, 1))


# speedup vs baseline: 1.1850x; 1.0138x over previous
import functools

import jax
import jax.numpy as jnp
from jax import lax
from jax.experimental import pallas as pl
from jax.experimental.pallas import tpu as pltpu
from jax.experimental.pallas import tpu_sc as plsc

L = 16
SB = 128
VB = 2048


def _split_point(V):
    return ((V // 2 + VB - 1) // VB) * VB


def _build_tc_pair_transpose(V, D):
    SP = _split_point(V)
    n_blk = SP // VB

    def body(t1_ref, t2_ref, o_ref):
        a = t1_ref[...]
        b = t2_ref[...]
        o_ref[...] = jnp.concatenate([a.T, b.T], axis=1)

    return pl.pallas_call(
        body,
        grid=(n_blk,),
        in_specs=[pl.BlockSpec((D, VB), lambda i: (0, i)),
                  pl.BlockSpec((D, VB), lambda i, nb=n_blk: (0, i + nb))],
        out_specs=pl.BlockSpec((VB, 2 * D), lambda i: (i, 0)),
        out_shape=jax.ShapeDtypeStruct((SP, 2 * D), jnp.float32),
    )


def _build_sc_gather(B, F, D, V, NW):
    BW = B // NW
    S = BW // SB
    G = SB // L
    DG = D // L
    NK = F * S
    SP = _split_point(V)
    assert BW * NW == B and S * SB == BW and G * L == SB and DG * L == D
    assert NK % 2 == 0 and V % 2 == 0

    mesh = plsc.VectorSubcoreMesh(core_axis_name="c", subcore_axis_name="s")

    @functools.partial(
        pl.kernel,
        mesh=mesh,
        compiler_params=pltpu.CompilerParams(
            needs_layout_passes=False, use_tc_tiling_on_sc=True),
        out_type=jax.ShapeDtypeStruct((F, D, B), jnp.float32),
        scratch_types=[
            pltpu.VMEM((SB,), jnp.int32),
            pltpu.VMEM((SB,), jnp.int32),
            pltpu.VMEM((SB,), jnp.int32),
            pltpu.VMEM((SB,), jnp.int32),
            pltpu.VMEM((F * L,), jnp.int32),
            pltpu.VMEM((SB, 2 * D), jnp.float32),
            pltpu.VMEM((SB, 2 * D), jnp.float32),
            pltpu.VMEM((L, SB), jnp.float32),
            pltpu.SemaphoreType.DMA,
            pltpu.SemaphoreType.DMA,
        ],
    )
    def sc_kernel(ptable_hbm, idx_hbm, off_hbm, out_hbm,
                  pv0, pv1, hb0, hb1, off_v, buf0, buf1, strip, sem0, sem1):
        wid = lax.axis_index("s") * 2 + lax.axis_index("c")
        b0 = wid * BW

        pltpu.sync_copy(off_hbm, off_v)

        qi = lax.iota(jnp.int32, L)

        def issue_block(k, pv, hb, buf, sem):
            f = k // S
            base = b0 + (k % S) * SB
            pltpu.sync_copy(idx_hbm.at[f, pl.ds(base, SB)], pv)
            ov = off_v[pl.ds(f * L, L)]

            def gbody(g, _):
                vv = pv[pl.ds(g * L, L)] + ov
                fold = jnp.where(vv >= SP, 1, 0)
                pv[pl.ds(g * L, L)] = vv - fold * SP
                hb[pl.ds(g * L, L)] = fold * D
                return 0
            lax.fori_loop(0, G, gbody, 0)
            pltpu.async_copy(ptable_hbm.at[pv], buf, sem)

        def drain(buf, sem):
            pltpu.make_async_copy(ptable_hbm.at[pl.ds(0, SB)], buf, sem).wait()

        def emit_block(k, hb, buf):
            f = k // S
            base = b0 + (k % S) * SB

            def sbody(dg, _):
                def gbody(g, _):
                    rows = qi + g * L
                    cols = hb[pl.ds(g * L, L)] + dg * L
                    for dl in range(L):
                        gvec = plsc.load_gather(buf, [rows, cols + dl])
                        strip[dl, pl.ds(g * L, L)] = gvec
                    return 0
                lax.fori_loop(0, G, gbody, 0)
                pltpu.sync_copy(strip,
                                out_hbm.at[f, pl.ds(dg * L, L), pl.ds(base, SB)])
                return 0
            lax.fori_loop(0, DG, sbody, 0)

        pvs = (pv0, pv1)
        hbs = (hb0, hb1)
        bufs = (buf0, buf1)
        sems = (sem0, sem1)

        issue_block(0, pvs[0], hbs[0], bufs[0], sems[0])

        def fbody(j, _):
            for p in range(2):
                k = j * 2 + p

                @pl.when(k + 1 < NK)
                def _():
                    issue_block(k + 1, pvs[1 - p], hbs[1 - p],
                                bufs[1 - p], sems[1 - p])
                drain(bufs[p], sems[p])
                emit_block(k, hbs[p], bufs[p])
            return 0

        lax.fori_loop(0, NK // 2, fbody, 0)

    return sc_kernel


def kernel(categorical_inputs, table, offsets):
    B, F = categorical_inputs.shape
    V, D = table.shape
    NW = 32

    tableT = jnp.transpose(table)
    idxT = jnp.transpose(categorical_inputs).astype(jnp.int32)
    off = jnp.repeat(offsets[:F].astype(jnp.int32), L)

    ptable = _build_tc_pair_transpose(V, D)(tableT, tableT)
    sc = _build_sc_gather(B, F, D, V, NW)
    outP = sc(ptable, idxT, off)
    return jnp.transpose(outP, (2, 0, 1))


# speedup vs baseline: 1.1867x; 1.0015x over previous
import functools

import jax
import jax.numpy as jnp
from jax import lax
from jax.experimental import pallas as pl
from jax.experimental.pallas import tpu as pltpu
from jax.experimental.pallas import tpu_sc as plsc

L = 16
SB = 128
VB = 2048


def _split_point(V):
    return ((V // 2 + VB - 1) // VB) * VB


def _build_tc_pair_transpose(V, D):
    SP = _split_point(V)
    n_blk = SP // VB

    def body(t1_ref, t2_ref, o_ref):
        a = t1_ref[...]
        b = t2_ref[...]
        c = jnp.concatenate([a, b], axis=0)
        ident = jnp.asarray(
            (lax.broadcasted_iota(jnp.int32, (2 * D, 2 * D), 0)
             == lax.broadcasted_iota(jnp.int32, (2 * D, 2 * D), 1)),
            dtype=jnp.float32)
        o_ref[...] = lax.dot_general(
            c, ident, (((0,), (0,)), ((), ())),
            precision=lax.Precision.HIGHEST)

    return pl.pallas_call(
        body,
        grid=(n_blk,),
        in_specs=[pl.BlockSpec((D, VB), lambda i: (0, i)),
                  pl.BlockSpec((D, VB), lambda i, nb=n_blk: (0, i + nb))],
        out_specs=pl.BlockSpec((VB, 2 * D), lambda i: (i, 0)),
        out_shape=jax.ShapeDtypeStruct((SP, 2 * D), jnp.float32),
    )


def _build_sc_gather(B, F, D, V, NW):
    BW = B // NW
    S = BW // SB
    G = SB // L
    DG = D // L
    NK = F * S
    SP = _split_point(V)
    assert BW * NW == B and S * SB == BW and G * L == SB and DG * L == D
    assert NK % 2 == 0 and V % 2 == 0

    mesh = plsc.VectorSubcoreMesh(core_axis_name="c", subcore_axis_name="s")

    @functools.partial(
        pl.kernel,
        mesh=mesh,
        compiler_params=pltpu.CompilerParams(
            needs_layout_passes=False, use_tc_tiling_on_sc=True),
        out_type=jax.ShapeDtypeStruct((F, D, B), jnp.float32),
        scratch_types=[
            pltpu.VMEM((SB,), jnp.int32),
            pltpu.VMEM((SB,), jnp.int32),
            pltpu.VMEM((SB,), jnp.int32),
            pltpu.VMEM((SB,), jnp.int32),
            pltpu.VMEM((F * L,), jnp.int32),
            pltpu.VMEM((SB, 2 * D), jnp.float32),
            pltpu.VMEM((SB, 2 * D), jnp.float32),
            pltpu.VMEM((L, SB), jnp.float32),
            pltpu.VMEM((L, SB), jnp.float32),
            pltpu.SemaphoreType.DMA,
            pltpu.SemaphoreType.DMA,
            pltpu.SemaphoreType.DMA,
            pltpu.SemaphoreType.DMA,
        ],
    )
    def sc_kernel(ptable_hbm, idx_hbm, off_hbm, out_hbm,
                  pv0, pv1, hb0, hb1, off_v, buf0, buf1, strip0, strip1,
                  sem0, sem1, wsem0, wsem1):
        wid = lax.axis_index("s") * 2 + lax.axis_index("c")
        b0 = wid * BW

        pltpu.sync_copy(off_hbm, off_v)

        qi = lax.iota(jnp.int32, L)

        def issue_block(k, pv, hb, buf, sem):
            f = k // S
            base = b0 + (k % S) * SB
            pltpu.sync_copy(idx_hbm.at[f, pl.ds(base, SB)], pv)
            ov = off_v[pl.ds(f * L, L)]

            def gbody(g, _):
                vv = pv[pl.ds(g * L, L)] + ov
                fold = jnp.where(vv >= SP, 1, 0)
                pv[pl.ds(g * L, L)] = vv - fold * SP
                hb[pl.ds(g * L, L)] = fold * D
                return 0
            lax.fori_loop(0, G, gbody, 0)
            pltpu.async_copy(ptable_hbm.at[pv], buf, sem)

        def drain(buf, sem):
            pltpu.make_async_copy(ptable_hbm.at[pl.ds(0, SB)], buf, sem).wait()

        strips = (strip0, strip1)
        wsems = (wsem0, wsem1)

        def wdrain(sp):
            pltpu.make_async_copy(out_hbm.at[0, pl.ds(0, L), pl.ds(0, SB)],
                                  strips[sp], wsems[sp]).wait()

        def emit_block(k, hb, buf):
            f = k // S
            base = b0 + (k % S) * SB

            for dg in range(DG):
                sp = dg % 2
                if dg < 2:
                    @pl.when(k > 0)
                    def _():
                        wdrain(sp)
                else:
                    wdrain(sp)
                strip = strips[sp]

                def gbody(g, _):
                    rows = qi + g * L
                    cols = hb[pl.ds(g * L, L)] + dg * L
                    for dl in range(L):
                        gvec = plsc.load_gather(buf, [rows, cols + dl])
                        strip[dl, pl.ds(g * L, L)] = gvec
                    return 0
                lax.fori_loop(0, G, gbody, 0)
                pltpu.async_copy(
                    strip, out_hbm.at[f, pl.ds(dg * L, L), pl.ds(base, SB)],
                    wsems[sp])

        pvs = (pv0, pv1)
        hbs = (hb0, hb1)
        bufs = (buf0, buf1)
        sems = (sem0, sem1)

        issue_block(0, pvs[0], hbs[0], bufs[0], sems[0])

        def fbody(j, _):
            for p in range(2):
                k = j * 2 + p

                @pl.when(k + 1 < NK)
                def _():
                    issue_block(k + 1, pvs[1 - p], hbs[1 - p],
                                bufs[1 - p], sems[1 - p])
                drain(bufs[p], sems[p])
                emit_block(k, hbs[p], bufs[p])
            return 0

        lax.fori_loop(0, NK // 2, fbody, 0)
        wdrain(0)
        wdrain(1)

    return sc_kernel


def kernel(categorical_inputs, table, offsets):
    B, F = categorical_inputs.shape
    V, D = table.shape
    NW = 32

    tableT = jnp.transpose(table)
    idxT = jnp.transpose(categorical_inputs).astype(jnp.int32)
    off = jnp.repeat(offsets[:F].astype(jnp.int32), L)

    ptable = _build_tc_pair_transpose(V, D)(tableT, tableT)
    sc = _build_sc_gather(B, F, D, V, NW)
    outP = sc(ptable, idxT, off)
    return jnp.transpose(outP, (2, 0, 1))


# speedup vs baseline: 1.2316x; 1.0378x over previous
import functools

import jax
import jax.numpy as jnp
from jax import lax
from jax.experimental import pallas as pl
from jax.experimental.pallas import tpu as pltpu
from jax.experimental.pallas import tpu_sc as plsc

L = 16
SB = 128
VB = 2048


def _split_point(V):
    return ((V // 2 + VB - 1) // VB) * VB


def _build_tc_pair_transpose(V, D):
    SP = _split_point(V)
    n_blk = SP // VB

    def body(t1_ref, t2_ref, o_ref):
        a = t1_ref[...]
        b = t2_ref[...]
        o_ref[...] = jnp.concatenate([a.T, b.T], axis=1)

    return pl.pallas_call(
        body,
        grid=(n_blk,),
        in_specs=[pl.BlockSpec((D, VB), lambda i: (0, i)),
                  pl.BlockSpec((D, VB), lambda i, nb=n_blk: (0, i + nb))],
        out_specs=pl.BlockSpec((VB, 2 * D), lambda i: (i, 0)),
        out_shape=jax.ShapeDtypeStruct((SP, 2 * D), jnp.float32),
    )


def _build_sc_gather(B, F, D, V, NW):
    BW = B // NW
    S = BW // SB
    G = SB // L
    DG = D // L
    NK = F * S
    SP = _split_point(V)
    assert BW * NW == B and S * SB == BW and G * L == SB and DG * L == D
    assert NK % 2 == 0 and V % 2 == 0

    mesh = plsc.VectorSubcoreMesh(core_axis_name="c", subcore_axis_name="s")

    @functools.partial(
        pl.kernel,
        mesh=mesh,
        compiler_params=pltpu.CompilerParams(
            needs_layout_passes=False, use_tc_tiling_on_sc=True),
        out_type=jax.ShapeDtypeStruct((F, D, B), jnp.float32),
        scratch_types=[
            pltpu.VMEM((SB,), jnp.int32),
            pltpu.VMEM((SB,), jnp.int32),
            pltpu.VMEM((SB,), jnp.int32),
            pltpu.VMEM((SB,), jnp.int32),
            pltpu.VMEM((F * L,), jnp.int32),
            pltpu.VMEM((SB, 2 * D), jnp.float32),
            pltpu.VMEM((SB, 2 * D), jnp.float32),
            pltpu.VMEM((L, SB), jnp.float32),
            pltpu.VMEM((L, SB), jnp.float32),
            pltpu.SemaphoreType.DMA,
            pltpu.SemaphoreType.DMA,
            pltpu.SemaphoreType.DMA,
            pltpu.SemaphoreType.DMA,
        ],
    )
    def sc_kernel(ptable_hbm, idx_hbm, off_hbm, out_hbm,
                  pv0, pv1, hb0, hb1, off_v, buf0, buf1, strip0, strip1,
                  sem0, sem1, wsem0, wsem1):
        wid = lax.axis_index("s") * 2 + lax.axis_index("c")
        b0 = wid * BW

        pltpu.sync_copy(off_hbm, off_v)

        qi = lax.iota(jnp.int32, L)

        def issue_block(k, pv, hb, buf, sem):
            f = k // S
            base = b0 + (k % S) * SB
            pltpu.sync_copy(idx_hbm.at[f, pl.ds(base, SB)], pv)
            ov = off_v[pl.ds(f * L, L)]

            def gbody(g, _):
                vv = pv[pl.ds(g * L, L)] + ov
                fold = jnp.where(vv >= SP, 1, 0)
                pv[pl.ds(g * L, L)] = vv - fold * SP
                hb[pl.ds(g * L, L)] = fold * D
                return 0
            lax.fori_loop(0, G, gbody, 0)
            pltpu.async_copy(ptable_hbm.at[pv], buf, sem)

        def drain(buf, sem):
            pltpu.make_async_copy(ptable_hbm.at[pl.ds(0, SB)], buf, sem).wait()

        strips = (strip0, strip1)
        wsems = (wsem0, wsem1)

        def wdrain(sp):
            pltpu.make_async_copy(out_hbm.at[0, pl.ds(0, L), pl.ds(0, SB)],
                                  strips[sp], wsems[sp]).wait()

        def emit_block(k, hb, buf):
            f = k // S
            base = b0 + (k % S) * SB

            for dg in range(DG):
                sp = dg % 2
                if dg < 2:
                    @pl.when(k > 0)
                    def _():
                        wdrain(sp)
                else:
                    wdrain(sp)
                strip = strips[sp]

                for g in range(G):
                    rows = qi + g * L
                    cols = hb[pl.ds(g * L, L)] + dg * L
                    for dl in range(L):
                        gvec = plsc.load_gather(buf, [rows, cols + dl])
                        strip[dl, pl.ds(g * L, L)] = gvec
                pltpu.async_copy(
                    strip, out_hbm.at[f, pl.ds(dg * L, L), pl.ds(base, SB)],
                    wsems[sp])

        pvs = (pv0, pv1)
        hbs = (hb0, hb1)
        bufs = (buf0, buf1)
        sems = (sem0, sem1)

        issue_block(0, pvs[0], hbs[0], bufs[0], sems[0])

        def fbody(j, _):
            for p in range(2):
                k = j * 2 + p

                @pl.when(k + 1 < NK)
                def _():
                    issue_block(k + 1, pvs[1 - p], hbs[1 - p],
                                bufs[1 - p], sems[1 - p])
                drain(bufs[p], sems[p])
                emit_block(k, hbs[p], bufs[p])
            return 0

        lax.fori_loop(0, NK // 2, fbody, 0)
        wdrain(0)
        wdrain(1)

    return sc_kernel


def kernel(categorical_inputs, table, offsets):
    B, F = categorical_inputs.shape
    V, D = table.shape
    NW = 32

    tableT = jnp.transpose(table)
    idxT = jnp.transpose(categorical_inputs).astype(jnp.int32)
    off = jnp.repeat(offsets[:F].astype(jnp.int32), L)

    ptable = _build_tc_pair_transpose(V, D)(tableT, tableT)
    sc = _build_sc_gather(B, F, D, V, NW)
    outP = sc(ptable, idxT, off)
    return jnp.transpose(outP, (2, 0, 1))


# speedup vs baseline: 1.5585x; 1.2654x over previous
import functools

import jax
import jax.numpy as jnp
from jax import lax
from jax.experimental import pallas as pl
from jax.experimental.pallas import tpu as pltpu
from jax.experimental.pallas import tpu_sc as plsc

L = 16
VB = 2048


def _split_point(V):
    return ((V // 2 + VB - 1) // VB) * VB


def _build_tc_pair_transpose(V, D):
    SP = _split_point(V)
    n_blk = SP // VB

    def body(t1_ref, t2_ref, o_ref):
        a = t1_ref[...]
        b = t2_ref[...]
        o_ref[...] = jnp.concatenate([a.T, b.T], axis=1)

    return pl.pallas_call(
        body,
        grid=(n_blk,),
        in_specs=[pl.BlockSpec((D, VB), lambda i: (0, i)),
                  pl.BlockSpec((D, VB), lambda i, nb=n_blk: (0, i + nb))],
        out_specs=pl.BlockSpec((VB, 2 * D), lambda i: (i, 0)),
        out_shape=jax.ShapeDtypeStruct((SP, 2 * D), jnp.float32),
    )


def _build_sc_gather(N, V, D, NW):
    per_w = N // NW
    R = 128
    n_chunks = per_w // R
    SP = _split_point(V)
    assert per_w * NW == N and n_chunks * R == per_w and n_chunks % 2 == 0

    mesh = plsc.VectorSubcoreMesh(core_axis_name="c", subcore_axis_name="s")

    @functools.partial(
        pl.kernel,
        mesh=mesh,
        out_type=jax.ShapeDtypeStruct((N, D), jnp.float32),
        compiler_params=pltpu.CompilerParams(use_tc_tiling_on_sc=False),
        scratch_types=[
            pltpu.VMEM((per_w,), jnp.int32),
            pltpu.VMEM((per_w,), jnp.int32),
            pltpu.VMEM((R, D), jnp.float32),
            pltpu.VMEM((R, D), jnp.float32),
            pltpu.SemaphoreType.DMA,
            pltpu.SemaphoreType.DMA,
        ],
    )
    def sc_kernel(idx_hbm, table_hbm, pat_hbm, out_hbm,
                  idx_v, pat_v, buf0, buf1, sem0, sem1):
        wid = lax.axis_index("s") * 2 + lax.axis_index("c")
        base = wid * per_w

        pltpu.sync_copy(idx_hbm.at[pl.ds(base, per_w)], idx_v)
        pltpu.sync_copy(pat_hbm, pat_v)

        def add_body(i, _):
            sl = pl.ds(i * L, L)
            vv = idx_v[sl] + pat_v[sl]
            fold = jnp.where(vv >= SP, 1, 0)
            idx_v[sl] = 2 * vv - fold * (2 * SP - 1)
            return 0

        lax.fori_loop(0, per_w // L, add_body, 0)

        def gather(c, buf, sem):
            pltpu.async_copy(table_hbm.at[idx_v.at[pl.ds(c * R, R)]], buf, sem)

        def gwait(buf, sem):
            pltpu.make_async_copy(table_hbm.at[pl.ds(0, R)], buf, sem).wait()

        gather(0, buf0, sem0)

        def chunk_body(j, _):
            c0 = 2 * j
            gather(c0 + 1, buf1, sem1)
            gwait(buf0, sem0)
            pltpu.sync_copy(buf0, out_hbm.at[pl.ds(base + c0 * R, R)])

            @pl.when(j + 1 < n_chunks // 2)
            def _():
                gather(c0 + 2, buf0, sem0)

            gwait(buf1, sem1)
            pltpu.sync_copy(buf1, out_hbm.at[pl.ds(base + (c0 + 1) * R, R)])
            return 0

        lax.fori_loop(0, n_chunks // 2, chunk_body, 0)

    return sc_kernel


def kernel(categorical_inputs, table, offsets):
    B, F = categorical_inputs.shape
    V, D = table.shape
    N = B * F
    NW = 32
    SP = _split_point(V)

    tableT = jnp.transpose(table)
    ptable = _build_tc_pair_transpose(V, D)(tableT, tableT)
    table2 = ptable.reshape(2 * SP, D)

    idx = categorical_inputs.astype(jnp.int32).reshape(N)
    per_w = N // NW
    pat = jnp.tile(offsets[:F].astype(jnp.int32), per_w // F)

    sc = _build_sc_gather(N, V, D, NW)
    out = sc(idx, table2, pat)
    return out.reshape(B, F, D)


# speedup vs baseline: 1.6481x; 1.0575x over previous
import functools

import jax
import jax.numpy as jnp
from jax import lax
from jax.experimental import pallas as pl
from jax.experimental.pallas import tpu as pltpu
from jax.experimental.pallas import tpu_sc as plsc

L = 16
VB = 2048


def _split_point(V):
    return ((V // 2 + VB - 1) // VB) * VB


def _build_tc_pair_transpose(V, D):
    SP = _split_point(V)
    n_blk = SP // VB

    def body(t1_ref, t2_ref, o_ref):
        a = t1_ref[...]
        b = t2_ref[...]
        c = jnp.concatenate([a, b], axis=0)
        ident = jnp.asarray(
            (lax.broadcasted_iota(jnp.int32, (2 * D, 2 * D), 0)
             == lax.broadcasted_iota(jnp.int32, (2 * D, 2 * D), 1)),
            dtype=jnp.bfloat16)
        hi = c.astype(jnp.bfloat16)
        r1 = c - hi.astype(jnp.float32)
        mid = r1.astype(jnp.bfloat16)
        lo = (r1 - mid.astype(jnp.float32)).astype(jnp.bfloat16)
        dn = (((0,), (0,)), ((), ()))

        def tdot(x):
            return lax.dot_general(x, ident, dn,
                                   preferred_element_type=jnp.float32)

        o_ref[...] = (tdot(hi) + tdot(mid)) + tdot(lo)

    return pl.pallas_call(
        body,
        grid=(n_blk,),
        in_specs=[pl.BlockSpec((D, VB), lambda i: (0, i)),
                  pl.BlockSpec((D, VB), lambda i, nb=n_blk: (0, i + nb))],
        out_specs=pl.BlockSpec((VB, 2 * D), lambda i: (i, 0)),
        out_shape=jax.ShapeDtypeStruct((SP, 2 * D), jnp.float32),
    )


def _build_sc_gather(N, V, D, NW):
    per_w = N // NW
    R = 128
    n_chunks = per_w // R
    SP = _split_point(V)
    assert per_w * NW == N and n_chunks * R == per_w and n_chunks % 2 == 0

    mesh = plsc.VectorSubcoreMesh(core_axis_name="c", subcore_axis_name="s")

    @functools.partial(
        pl.kernel,
        mesh=mesh,
        out_type=jax.ShapeDtypeStruct((N, D), jnp.float32),
        compiler_params=pltpu.CompilerParams(use_tc_tiling_on_sc=False),
        scratch_types=[
            pltpu.VMEM((per_w,), jnp.int32),
            pltpu.VMEM((per_w,), jnp.int32),
            pltpu.VMEM((R, D), jnp.float32),
            pltpu.VMEM((R, D), jnp.float32),
            pltpu.SemaphoreType.DMA,
            pltpu.SemaphoreType.DMA,
        ],
    )
    def sc_kernel(idx_hbm, table_hbm, pat_hbm, out_hbm,
                  idx_v, pat_v, buf0, buf1, sem0, sem1):
        wid = lax.axis_index("s") * 2 + lax.axis_index("c")
        base = wid * per_w

        pltpu.sync_copy(idx_hbm.at[pl.ds(base, per_w)], idx_v)
        pltpu.sync_copy(pat_hbm, pat_v)

        def add_body(i, _):
            sl = pl.ds(i * L, L)
            vv = idx_v[sl] + pat_v[sl]
            fold = jnp.where(vv >= SP, 1, 0)
            idx_v[sl] = 2 * vv - fold * (2 * SP - 1)
            return 0

        lax.fori_loop(0, per_w // L, add_body, 0)

        def gather(c, buf, sem):
            pltpu.async_copy(table_hbm.at[idx_v.at[pl.ds(c * R, R)]], buf, sem)

        def gwait(buf, sem):
            pltpu.make_async_copy(table_hbm.at[pl.ds(0, R)], buf, sem).wait()

        gather(0, buf0, sem0)

        def chunk_body(j, _):
            c0 = 2 * j
            gather(c0 + 1, buf1, sem1)
            gwait(buf0, sem0)
            pltpu.sync_copy(buf0, out_hbm.at[pl.ds(base + c0 * R, R)])

            @pl.when(j + 1 < n_chunks // 2)
            def _():
                gather(c0 + 2, buf0, sem0)

            gwait(buf1, sem1)
            pltpu.sync_copy(buf1, out_hbm.at[pl.ds(base + (c0 + 1) * R, R)])
            return 0

        lax.fori_loop(0, n_chunks // 2, chunk_body, 0)

    return sc_kernel


def kernel(categorical_inputs, table, offsets):
    B, F = categorical_inputs.shape
    V, D = table.shape
    N = B * F
    NW = 32
    SP = _split_point(V)

    tableT = jnp.transpose(table)
    ptable = _build_tc_pair_transpose(V, D)(tableT, tableT)
    table2 = ptable.reshape(2 * SP, D)

    idx = categorical_inputs.astype(jnp.int32).reshape(N)
    per_w = N // NW
    pat = jnp.tile(offsets[:F].astype(jnp.int32), per_w // F)

    sc = _build_sc_gather(N, V, D, NW)
    out = sc(idx, table2, pat)
    return out.reshape(B, F, D)
